# Initial kernel scaffold; baseline (speedup 1.0000x reference)
import jax
import jax.numpy as jnp
from jax.experimental import pallas as pl


def kernel(X1, X2, clebsch):
    raise NotImplementedError("write your pallas kernel here")


# speedup vs baseline: 7.0127x; 7.0127x over previous
import jax
import jax.numpy as jnp
from jax.experimental import pallas as pl
from jax.experimental.pallas import tpu as pltpu

LAMBD_M = 9
TILE_N = 1000


def _combine_kernel(c_ref, x1_ref, x2_ref, o_ref):
    for k in range(LAMBD_M):
        acc = None
        for m1 in range(k + 1):
            m2 = k - m1
            term = x1_ref[m1] * x2_ref[m2] * c_ref[m1, m2]
            acc = term if acc is None else acc + term
        o_ref[k] = acc


def kernel(X1, X2, clebsch):
    m1s, n, f = X1.shape
    grid = (n // TILE_N,)
    return pl.pallas_call(
        _combine_kernel,
        grid=grid,
        in_specs=[
            pl.BlockSpec(memory_space=pltpu.SMEM),
            pl.BlockSpec((m1s, TILE_N, f), lambda i: (0, i, 0)),
            pl.BlockSpec((m1s, TILE_N, f), lambda i: (0, i, 0)),
        ],
        out_specs=pl.BlockSpec((LAMBD_M, TILE_N, f), lambda i: (0, i, 0)),
        out_shape=jax.ShapeDtypeStruct((LAMBD_M, n, f), X1.dtype),
        compiler_params=pltpu.CompilerParams(
            dimension_semantics=("arbitrary",),
        ),
    )(clebsch, X1, X2)
